# Initial kernel scaffold; baseline (speedup 1.0000x reference)
import jax
import jax.numpy as jnp
from jax.experimental import pallas as pl


def kernel(node_attr, edge_index, edge_attr, batch, W_msg, b_msg, W1, b1, W2, b2, W3, b3, W4, b4, W5, b5):
    raise NotImplementedError("write your pallas kernel here")


# speedup vs baseline: 6.0915x; 6.0915x over previous
import functools

import jax
import jax.numpy as jnp
from jax import lax
from jax.experimental import pallas as pl
from jax.experimental.pallas import tpu as pltpu
from jax.experimental.pallas import tpu_sc as plsc

N_NODES = 10000
N_EDGES = 320000
D_FEAT = 128
D_EDGE = 16
HID_PAD = 16
N_GRAPHS = 64

NC, NS = 2, 16
NW = NC * NS
CHUNK = 128
CHUNKS_PER_W = 80
EDGES_PER_W = CHUNK * CHUNKS_PER_W
E_PAD = EDGES_PER_W * NW
N_PAD = 10240
ROWS_PER_TILE = N_PAD // NS
NEG = -1e9


_N8 = N_NODES // 8


def _proj_body(x_ref, ws_ref, wd_ref, a_ref, b_ref):
    x = x_ref[...]
    a_ref[...] = jnp.dot(x, ws_ref[...], preferred_element_type=jnp.float32)
    b_ref[...] = jnp.dot(x, wd_ref[...], preferred_element_type=jnp.float32)


def _node_proj(na8, ws_big, wd_big):
    return pl.pallas_call(
        _proj_body,
        grid=(1,),
        in_specs=[
            pl.BlockSpec((_N8, 8 * D_FEAT), lambda i: (0, 0)),
            pl.BlockSpec((8 * D_FEAT, 128), lambda i: (0, 0)),
            pl.BlockSpec((8 * D_FEAT, 128), lambda i: (0, 0)),
        ],
        out_specs=[
            pl.BlockSpec((_N8, 128), lambda i: (0, 0)),
            pl.BlockSpec((_N8, 128), lambda i: (0, 0)),
        ],
        out_shape=[
            jax.ShapeDtypeStruct((_N8, 128), jnp.float32),
            jax.ShapeDtypeStruct((_N8, 128), jnp.float32),
        ],
    )(na8, ws_big, wd_big)


_E8 = N_EDGES // 8
_E8_PAD = E_PAD // 8
_EBLK = 320
_N_REAL_EBLK = _E8 // _EBLK
_N_EBLK = _E8_PAD // _EBLK


def _edge_pre_body(ea_ref, we_ref, bm_ref, e_ref):
    i = pl.program_id(0)

    @pl.when(i < _N_REAL_EBLK)
    def _():
        e_ref[...] = (
            jnp.dot(ea_ref[...], we_ref[...], preferred_element_type=jnp.float32)
            + bm_ref[...]
        )

    @pl.when(i >= _N_REAL_EBLK)
    def _():
        e_ref[...] = jnp.full((_EBLK, 128), NEG, dtype=jnp.float32)


def _edge_pre(ea8, we_big, bm_big):
    return pl.pallas_call(
        _edge_pre_body,
        grid=(_N_EBLK,),
        in_specs=[
            pl.BlockSpec(
                (_EBLK, 8 * D_EDGE),
                lambda i: (jnp.minimum(i, _N_REAL_EBLK - 1), 0),
            ),
            pl.BlockSpec((8 * D_EDGE, 128), lambda i: (0, 0)),
            pl.BlockSpec((1, 128), lambda i: (0, 0)),
        ],
        out_specs=pl.BlockSpec((_EBLK, 128), lambda i: (i, 0)),
        out_shape=jax.ShapeDtypeStruct((_E8_PAD, 128), jnp.float32),
    )(ea8, we_big, bm_big)


def _sc_edge_body(a_hbm, b_hbm, e_hbm, src_hbm, dst_hbm, out_hbm,
                  sidx_v, didx_v, a_v, b_v, e_v, msg_v, zbuf, sem_a, sem_b,
                  sem_e, acc):
    cid = lax.axis_index("c")
    sid = lax.axis_index("s")
    wid = sid * NC + cid

    def _zrow(i, _):
        zbuf[i] = jnp.zeros((HID_PAD,), jnp.float32)
        return 0

    lax.fori_loop(0, CHUNK, _zrow, 0)
    n_zcopies = ROWS_PER_TILE // CHUNK
    for k in range(n_zcopies):
        pltpu.sync_copy(zbuf, acc.at[pl.ds(sid * ROWS_PER_TILE + k * CHUNK, CHUNK)])
    plsc.subcore_barrier()

    def _chunk(g, _):
        base = wid * EDGES_PER_W + g * CHUNK
        pltpu.sync_copy(src_hbm.at[pl.ds(base, CHUNK)], sidx_v)
        pltpu.sync_copy(dst_hbm.at[pl.ds(base, CHUNK)], didx_v)
        cp_a = pltpu.async_copy(a_hbm.at[sidx_v], a_v, sem_a)
        cp_b = pltpu.async_copy(b_hbm.at[didx_v], b_v, sem_b)
        cp_e = pltpu.async_copy(e_hbm.at[pl.ds(base, CHUNK)], e_v, sem_e)
        cp_a.wait()
        cp_b.wait()
        cp_e.wait()

        def _row(i, _):
            v = a_v[i] + b_v[i] + e_v[i]
            msg_v[i] = jnp.maximum(v, 0.0)
            return 0

        lax.fori_loop(0, CHUNK, _row, 0)
        pltpu.sync_copy(msg_v, acc.at[didx_v], add=True)
        return 0

    lax.fori_loop(0, CHUNKS_PER_W, _chunk, 0)
    plsc.subcore_barrier()

    pltpu.sync_copy(
        acc.at[pl.ds(sid * ROWS_PER_TILE, ROWS_PER_TILE)],
        out_hbm.at[cid, pl.ds(sid * ROWS_PER_TILE, ROWS_PER_TILE)],
    )


def _sc_edge(a, b, e, src, dst):
    mesh = plsc.VectorSubcoreMesh(core_axis_name="c", subcore_axis_name="s")
    fn = pl.kernel(
        _sc_edge_body,
        out_type=jax.ShapeDtypeStruct((NC, N_PAD, HID_PAD), jnp.float32),
        mesh=mesh,
        scratch_types=[
            pltpu.VMEM((CHUNK,), jnp.int32),
            pltpu.VMEM((CHUNK,), jnp.int32),
            pltpu.VMEM((CHUNK, HID_PAD), jnp.float32),
            pltpu.VMEM((CHUNK, HID_PAD), jnp.float32),
            pltpu.VMEM((CHUNK, HID_PAD), jnp.float32),
            pltpu.VMEM((CHUNK, HID_PAD), jnp.float32),
            pltpu.VMEM((CHUNK, HID_PAD), jnp.float32),
            pltpu.SemaphoreType.DMA,
            pltpu.SemaphoreType.DMA,
            pltpu.SemaphoreType.DMA,
            pltpu.VMEM_SHARED((N_PAD, HID_PAD), jnp.float32),
        ],
        compiler_params=pltpu.CompilerParams(use_tc_tiling_on_sc=False),
    )
    return fn(a, b, e, src, dst)


_POOL_BLK = 1024
_N_PBLK = N_PAD // _POOL_BLK


def _post_body(xp_ref, batch_ref, w1_ref, b1_ref, w2_ref, b2_ref, w3_ref,
               b3_ref, w4_ref, b4_ref, w5_ref, b5_ref, out_ref, g_acc):
    i = pl.program_id(0)
    x = xp_ref[0] + xp_ref[1]
    h = jnp.maximum(jnp.dot(x, w1_ref[...], preferred_element_type=jnp.float32)
                    + b1_ref[...], 0.0)
    h = jnp.maximum(jnp.dot(h, w2_ref[...], preferred_element_type=jnp.float32)
                    + b2_ref[...], 0.0)
    h = jnp.maximum(jnp.dot(h, w3_ref[...], preferred_element_type=jnp.float32)
                    + b3_ref[...], 0.0)
    bvec = batch_ref[0, 0, :]
    onehot = (bvec[:, None]
              == lax.broadcasted_iota(jnp.int32, (_POOL_BLK, N_GRAPHS), 1))
    part = lax.dot_general(
        onehot.astype(jnp.float32), h,
        (((0,), (0,)), ((), ())),
        preferred_element_type=jnp.float32,
    )

    @pl.when(i == 0)
    def _():
        g_acc[...] = part

    @pl.when(i > 0)
    def _():
        g_acc[...] = g_acc[...] + part

    @pl.when(i == _N_PBLK - 1)
    def _():
        g = jnp.maximum(
            jnp.dot(g_acc[...], w4_ref[...], preferred_element_type=jnp.float32)
            + b4_ref[...], 0.0)
        out_ref[...] = (
            jnp.dot(g, w5_ref[...], preferred_element_type=jnp.float32)
            + b5_ref[...])


def _post(xp, batch3, w1, b1, w2, b2, w3, b3, w4, b4, w5, b5):
    wspec = pl.BlockSpec((HID_PAD, HID_PAD), lambda i: (0, 0))
    bspec = pl.BlockSpec((1, HID_PAD), lambda i: (0, 0))
    return pl.pallas_call(
        _post_body,
        grid=(_N_PBLK,),
        in_specs=[
            pl.BlockSpec((NC, _POOL_BLK, HID_PAD), lambda i: (0, i, 0)),
            pl.BlockSpec((1, 1, _POOL_BLK), lambda i: (i, 0, 0)),
            wspec, bspec, wspec, bspec, wspec, bspec, wspec, bspec,
            wspec, bspec,
        ],
        out_specs=pl.BlockSpec((N_GRAPHS, HID_PAD), lambda i: (0, 0)),
        out_shape=jax.ShapeDtypeStruct((N_GRAPHS, HID_PAD), jnp.float32),
        scratch_shapes=[pltpu.VMEM((N_GRAPHS, HID_PAD), jnp.float32)],
    )(xp, batch3, w1, b1, w2, b2, w3, b3, w4, b4, w5, b5)


def _pad_w(w, rows, cols):
    out = jnp.zeros((rows, cols), jnp.float32)
    return out.at[: w.shape[0], : w.shape[1]].set(w)


def _pad_b(b):
    out = jnp.zeros((1, HID_PAD), jnp.float32)
    return out.at[0, : b.shape[0]].set(b)


def kernel(node_attr, edge_index, edge_attr, batch,
           W_msg, b_msg, W1, b1, W2, b2, W3, b3, W4, b4, W5, b5):
    src = edge_index[0].astype(jnp.int32)
    dst = edge_index[1].astype(jnp.int32)
    pad = jnp.zeros((E_PAD - N_EDGES,), jnp.int32)
    src = jnp.concatenate([src, pad])
    dst = jnp.concatenate([dst, pad])

    eye8 = jnp.eye(8, dtype=jnp.float32)
    ws_big = jnp.kron(eye8, _pad_w(W_msg[:D_FEAT], D_FEAT, HID_PAD))
    wd_big = jnp.kron(eye8, _pad_w(W_msg[D_FEAT:2 * D_FEAT], D_FEAT, HID_PAD))
    we_big = jnp.kron(eye8, _pad_w(W_msg[2 * D_FEAT:], D_EDGE, HID_PAD))
    bm_big = jnp.tile(_pad_b(b_msg), (1, 8))

    na8 = node_attr.reshape(_N8, 8 * D_FEAT)
    ea8 = edge_attr.reshape(_E8, 8 * D_EDGE)
    a_pack, b_pack = _node_proj(na8, ws_big, wd_big)
    e_pack = _edge_pre(ea8, we_big, bm_big)
    xp = _sc_edge(
        a_pack.reshape(N_NODES, HID_PAD),
        b_pack.reshape(N_NODES, HID_PAD),
        e_pack.reshape(E_PAD, HID_PAD),
        src, dst,
    )

    batch_pad = jnp.concatenate([
        batch.astype(jnp.int32),
        jnp.full((N_PAD - N_NODES,), N_GRAPHS, jnp.int32),
    ])
    batch3 = batch_pad.reshape(_N_PBLK, 1, _POOL_BLK)
    out16 = _post(
        xp, batch3,
        _pad_w(W1, HID_PAD, HID_PAD), _pad_b(b1),
        _pad_w(W2, HID_PAD, HID_PAD), _pad_b(b2),
        _pad_w(W3, HID_PAD, HID_PAD), _pad_b(b3),
        _pad_w(W4, HID_PAD, HID_PAD), _pad_b(b4),
        _pad_w(W5, HID_PAD, HID_PAD), _pad_b(b5),
    )
    return out16[:, :1]


# speedup vs baseline: 10.6835x; 1.7538x over previous
import functools

import jax
import jax.numpy as jnp
from jax import lax
from jax.experimental import pallas as pl
from jax.experimental.pallas import tpu as pltpu
from jax.experimental.pallas import tpu_sc as plsc

N_NODES = 10000
N_EDGES = 320000
D_FEAT = 128
D_EDGE = 16
HID_PAD = 16
N_GRAPHS = 64

NC, NS = 2, 16
NW = NC * NS
SUPER = 512
N_SUPER = N_EDGES // SUPER
Q_MAX = (N_SUPER + NW - 1) // NW
N_PAD = 10240
ROWS_PER_TILE = N_PAD // NS


_N8 = N_NODES // 8


def _proj_body(x_ref, ws_ref, wd_ref, a_ref, b_ref):
    x = x_ref[...]
    a_ref[...] = jnp.dot(x, ws_ref[...], preferred_element_type=jnp.float32)
    b_ref[...] = jnp.dot(x, wd_ref[...], preferred_element_type=jnp.float32)


def _node_proj(na8, ws_big, wd_big):
    return pl.pallas_call(
        _proj_body,
        grid=(1,),
        in_specs=[
            pl.BlockSpec((_N8, 8 * D_FEAT), lambda i: (0, 0)),
            pl.BlockSpec((8 * D_FEAT, 128), lambda i: (0, 0)),
            pl.BlockSpec((8 * D_FEAT, 128), lambda i: (0, 0)),
        ],
        out_specs=[
            pl.BlockSpec((_N8, 128), lambda i: (0, 0)),
            pl.BlockSpec((_N8, 128), lambda i: (0, 0)),
        ],
        out_shape=[
            jax.ShapeDtypeStruct((_N8, 128), jnp.float32),
            jax.ShapeDtypeStruct((_N8, 128), jnp.float32),
        ],
    )(na8, ws_big, wd_big)


_E8 = N_EDGES // 8
_EBLK = 2000
_N_EBLK = _E8 // _EBLK


def _edge_pre_body(ea_ref, we_ref, bm_ref, e_ref):
    e_ref[...] = (
        jnp.dot(ea_ref[...], we_ref[...], preferred_element_type=jnp.float32)
        + bm_ref[...]
    )


def _edge_pre(ea8, we_big, bm_big):
    return pl.pallas_call(
        _edge_pre_body,
        grid=(_N_EBLK,),
        in_specs=[
            pl.BlockSpec((_EBLK, 8 * D_EDGE), lambda i: (i, 0)),
            pl.BlockSpec((8 * D_EDGE, 128), lambda i: (0, 0)),
            pl.BlockSpec((1, 128), lambda i: (0, 0)),
        ],
        out_specs=pl.BlockSpec((_EBLK, 128), lambda i: (i, 0)),
        out_shape=jax.ShapeDtypeStruct((_E8, 128), jnp.float32),
    )(ea8, we_big, bm_big)


def _sc_edge_body(a_hbm, b_hbm, e_hbm, src_hbm, dst_hbm, out_hbm,
                  sidx_v, didx_v, a_v, b_v, e_v, msg_v,
                  sem_i, sem_g, sem_e, sem_s, acc):
    cid = lax.axis_index("c")
    sid = lax.axis_index("s")
    wid = sid * NC + cid

    def _zrow(i, _):
        msg_v[i] = jnp.zeros((HID_PAD,), jnp.float32)
        return 0

    lax.fori_loop(0, SUPER, _zrow, 0)
    pltpu.sync_copy(msg_v, acc.at[pl.ds(sid * ROWS_PER_TILE, SUPER)])
    pltpu.sync_copy(
        msg_v.at[pl.ds(0, ROWS_PER_TILE - SUPER)],
        acc.at[pl.ds(sid * ROWS_PER_TILE + SUPER, ROWS_PER_TILE - SUPER)],
    )
    plsc.subcore_barrier()

    def _super(q, _):
        s = q * NW + wid

        @pl.when(s < N_SUPER)
        def _():
            base = s * SUPER
            icps = []
            for k in range(4):
                icps.append(pltpu.async_copy(
                    src_hbm.at[pl.ds(base + k * 128, 128)], sidx_v.at[k],
                    sem_i))
                icps.append(pltpu.async_copy(
                    dst_hbm.at[pl.ds(base + k * 128, 128)], didx_v.at[k],
                    sem_i))
            for cp in icps:
                cp.wait()
            gcps = []
            for k in range(4):
                gcps.append(pltpu.async_copy(
                    a_hbm.at[sidx_v.at[k]], a_v.at[pl.ds(k * 128, 128)],
                    sem_g))
                gcps.append(pltpu.async_copy(
                    b_hbm.at[didx_v.at[k]], b_v.at[pl.ds(k * 128, 128)],
                    sem_g))
            ecp = pltpu.async_copy(e_hbm.at[pl.ds(base, SUPER)], e_v, sem_e)
            for cp in gcps:
                cp.wait()
            ecp.wait()

            @plsc.parallel_loop(0, SUPER, unroll=8)
            def _row(i):
                msg_v[i] = jnp.maximum(a_v[i] + b_v[i] + e_v[i], 0.0)

            scps = []
            for k in range(4):
                scps.append(pltpu.async_copy(
                    msg_v.at[pl.ds(k * 128, 128)], acc.at[didx_v.at[k]],
                    sem_s, add=True))
            for cp in scps:
                cp.wait()

        return 0

    lax.fori_loop(0, Q_MAX, _super, 0)
    plsc.subcore_barrier()

    pltpu.sync_copy(
        acc.at[pl.ds(sid * ROWS_PER_TILE, ROWS_PER_TILE)],
        out_hbm.at[cid, pl.ds(sid * ROWS_PER_TILE, ROWS_PER_TILE)],
    )


def _sc_edge(a, b, e, src, dst):
    mesh = plsc.VectorSubcoreMesh(core_axis_name="c", subcore_axis_name="s")
    fn = pl.kernel(
        _sc_edge_body,
        out_type=jax.ShapeDtypeStruct((NC, N_PAD, HID_PAD), jnp.float32),
        mesh=mesh,
        scratch_types=[
            pltpu.VMEM((4, 128), jnp.int32),
            pltpu.VMEM((4, 128), jnp.int32),
            pltpu.VMEM((SUPER, HID_PAD), jnp.float32),
            pltpu.VMEM((SUPER, HID_PAD), jnp.float32),
            pltpu.VMEM((SUPER, HID_PAD), jnp.float32),
            pltpu.VMEM((SUPER, HID_PAD), jnp.float32),
            pltpu.SemaphoreType.DMA,
            pltpu.SemaphoreType.DMA,
            pltpu.SemaphoreType.DMA,
            pltpu.SemaphoreType.DMA,
            pltpu.VMEM_SHARED((N_PAD, HID_PAD), jnp.float32),
        ],
        compiler_params=pltpu.CompilerParams(use_tc_tiling_on_sc=False),
    )
    return fn(a, b, e, src, dst)


_PBLK = 128
_N8_PAD = N_PAD // 8
_N_PBLK = _N8_PAD // _PBLK


def _post_body(xp_ref, bt_ref, w1_ref, b1_ref, w2_ref, b2_ref, w3_ref,
               b3_ref, w4_ref, b4_ref, w5_ref, b5_ref, out_ref, g_acc):
    i = pl.program_id(0)
    h = xp_ref[0] + xp_ref[1]
    h = jnp.maximum(jnp.dot(h, w1_ref[...], preferred_element_type=jnp.float32)
                    + b1_ref[...], 0.0)
    h = jnp.maximum(jnp.dot(h, w2_ref[...], preferred_element_type=jnp.float32)
                    + b2_ref[...], 0.0)
    h = jnp.maximum(jnp.dot(h, w3_ref[...], preferred_element_type=jnp.float32)
                    + b3_ref[...], 0.0)
    part = jnp.zeros((N_GRAPHS, HID_PAD), jnp.float32)
    iota_g = lax.broadcasted_iota(jnp.int32, (_PBLK, N_GRAPHS), 1)
    for j in range(8):
        bcol = bt_ref[j, :]
        oh = (bcol[:, None] == iota_g).astype(jnp.float32)
        m = lax.dot_general(oh, h, (((0,), (0,)), ((), ())),
                            preferred_element_type=jnp.float32)
        part = part + m[:, 16 * j:16 * j + 16]

    @pl.when(i == 0)
    def _():
        g_acc[...] = part

    @pl.when(i > 0)
    def _():
        g_acc[...] = g_acc[...] + part

    @pl.when(i == _N_PBLK - 1)
    def _():
        g = jnp.maximum(
            jnp.dot(g_acc[...], w4_ref[...], preferred_element_type=jnp.float32)
            + b4_ref[...], 0.0)
        out_ref[...] = (
            jnp.dot(g, w5_ref[...], preferred_element_type=jnp.float32)
            + b5_ref[...])


def _post(xp8, batch_t, w1, b1, w2, b2, w3, b3, w4, b4, w5, b5):
    wbig = pl.BlockSpec((128, 128), lambda i: (0, 0))
    bbig = pl.BlockSpec((1, 128), lambda i: (0, 0))
    wsmall = pl.BlockSpec((HID_PAD, HID_PAD), lambda i: (0, 0))
    bsmall = pl.BlockSpec((1, HID_PAD), lambda i: (0, 0))
    return pl.pallas_call(
        _post_body,
        grid=(_N_PBLK,),
        in_specs=[
            pl.BlockSpec((NC, _PBLK, 128), lambda i: (0, i, 0)),
            pl.BlockSpec((8, _PBLK), lambda i: (0, i)),
            wbig, bbig, wbig, bbig, wbig, bbig, wsmall, bsmall,
            wsmall, bsmall,
        ],
        out_specs=pl.BlockSpec((N_GRAPHS, HID_PAD), lambda i: (0, 0)),
        out_shape=jax.ShapeDtypeStruct((N_GRAPHS, HID_PAD), jnp.float32),
        scratch_shapes=[pltpu.VMEM((N_GRAPHS, HID_PAD), jnp.float32)],
    )(xp8, batch_t, w1, b1, w2, b2, w3, b3, w4, b4, w5, b5)


def _pad_w(w, rows, cols):
    out = jnp.zeros((rows, cols), jnp.float32)
    return out.at[: w.shape[0], : w.shape[1]].set(w)


def _pad_b(b):
    out = jnp.zeros((1, HID_PAD), jnp.float32)
    return out.at[0, : b.shape[0]].set(b)


def kernel(node_attr, edge_index, edge_attr, batch,
           W_msg, b_msg, W1, b1, W2, b2, W3, b3, W4, b4, W5, b5):
    src = edge_index[0].astype(jnp.int32)
    dst = edge_index[1].astype(jnp.int32)

    eye8 = jnp.eye(8, dtype=jnp.float32)

    def big_w(w, rows):
        return jnp.kron(eye8, _pad_w(w, rows, HID_PAD))

    def big_b(b):
        return jnp.tile(_pad_b(b), (1, 8))

    na8 = node_attr.reshape(_N8, 8 * D_FEAT)
    ea8 = edge_attr.reshape(_E8, 8 * D_EDGE)
    a_pack, b_pack = _node_proj(na8, big_w(W_msg[:D_FEAT], D_FEAT),
                                big_w(W_msg[D_FEAT:2 * D_FEAT], D_FEAT))
    e_pack = _edge_pre(ea8, big_w(W_msg[2 * D_FEAT:], D_EDGE), big_b(b_msg))
    xp = _sc_edge(
        a_pack.reshape(N_NODES, HID_PAD),
        b_pack.reshape(N_NODES, HID_PAD),
        e_pack.reshape(N_EDGES, HID_PAD),
        src, dst,
    )

    batch_t = jnp.concatenate([
        batch.astype(jnp.int32),
        jnp.full((N_PAD - N_NODES,), N_GRAPHS, jnp.int32),
    ]).reshape(_N8_PAD, 8).T
    out16 = _post(
        xp.reshape(NC, _N8_PAD, 128), batch_t,
        big_w(W1, HID_PAD), big_b(b1),
        big_w(W2, HID_PAD), big_b(b2),
        big_w(W3, HID_PAD), big_b(b3),
        _pad_w(W4, HID_PAD, HID_PAD), _pad_b(b4),
        _pad_w(W5, HID_PAD, HID_PAD), _pad_b(b5),
    )
    return out16[:, :1]
